# baseline (speedup 1.0000x reference)
import functools

import jax
import jax.numpy as jnp
from jax import lax
from jax.experimental import pallas as pl
from jax.experimental.pallas import tpu as pltpu
from jax.experimental.pallas import tpu_sc as plsc

N_ROWS = 1000000
D = 64
B = 16384
Y_HIGH = 5.5

NC = 2
NS = 16
L = 16
NW = NC * NS
BPW = B // NW
CHUNK = 128
NCHUNK = BPW // CHUNK
NBLK = BPW // L


def _sc_body(u_idx_hbm, m_idx_hbm, uf_hbm, ub_hbm, mf_hbm, mb_hbm, out_hbm,
             u_idx_v, m_idx_v, u_rows, m_rows, u_bias_v, m_bias_v,
             out_v, sem):
    wid = lax.axis_index("s") * NC + lax.axis_index("c")
    base = pl.multiple_of(wid * BPW, BPW)

    pltpu.sync_copy(u_idx_hbm.at[wid], u_idx_v)
    pltpu.sync_copy(m_idx_hbm.at[wid], m_idx_v)

    cps = []
    for j in range(NCHUNK):
        dst = pl.ds(j * CHUNK, CHUNK)
        cps.append(pltpu.async_copy(uf_hbm.at[u_idx_v.at[j]], u_rows.at[dst], sem))
        cps.append(pltpu.async_copy(mf_hbm.at[m_idx_v.at[j]], m_rows.at[dst], sem))
        cps.append(pltpu.async_copy(ub_hbm.at[u_idx_v.at[j]], u_bias_v.at[dst], sem))
        cps.append(pltpu.async_copy(mb_hbm.at[m_idx_v.at[j]], m_bias_v.at[dst], sem))
    for c in cps:
        c.wait()

    iota = lax.iota(jnp.int32, L)

    def block(b, carry):
        b16 = pl.multiple_of(b * L, L)
        acc = jnp.zeros((L,), jnp.float32)
        for i in range(L):
            r = b16 + i
            p0 = u_rows[r, pl.ds(0, 16)] * m_rows[r, pl.ds(0, 16)]
            p1 = u_rows[r, pl.ds(16, 16)] * m_rows[r, pl.ds(16, 16)]
            p2 = u_rows[r, pl.ds(32, 16)] * m_rows[r, pl.ds(32, 16)]
            p3 = u_rows[r, pl.ds(48, 16)] * m_rows[r, pl.ds(48, 16)]
            v = (p0 + p1) + (p2 + p3)
            s = jnp.sum(v)
            acc = jnp.where(iota == i, s, acc)
        t = acc + u_bias_v[pl.ds(b16, L)] + m_bias_v[pl.ds(b16, L)]
        out_v[pl.ds(b16, L)] = Y_HIGH / (1.0 + jnp.exp(-t))
        return carry

    lax.fori_loop(0, NBLK, block, 0)

    pltpu.sync_copy(out_v, out_hbm.at[pl.ds(base, BPW)])


@jax.jit
def _run(u_idx, m_idx, user_factors, user_bias, movie_factors, movie_bias):
    mesh = plsc.VectorSubcoreMesh(core_axis_name="c", subcore_axis_name="s")
    f = pl.kernel(
        _sc_body,
        mesh=mesh,
        compiler_params=pltpu.CompilerParams(
            needs_layout_passes=False, use_tc_tiling_on_sc=False),
        out_type=jax.ShapeDtypeStruct((B,), jnp.float32),
        scratch_types=[
            pltpu.VMEM((NCHUNK, CHUNK), jnp.int32),
            pltpu.VMEM((NCHUNK, CHUNK), jnp.int32),
            pltpu.VMEM((BPW, D), jnp.float32),
            pltpu.VMEM((BPW, D), jnp.float32),
            pltpu.VMEM((BPW,), jnp.float32),
            pltpu.VMEM((BPW,), jnp.float32),
            pltpu.VMEM((BPW,), jnp.float32),
            pltpu.SemaphoreType.DMA,
        ],
    )
    return f(u_idx, m_idx, user_factors, user_bias, movie_factors, movie_bias)


def kernel(x, user_factors, user_bias, movie_factors, movie_bias):
    u_idx = x[:, 0].reshape(NW, NCHUNK, CHUNK)
    m_idx = x[:, 1].reshape(NW, NCHUNK, CHUNK)
    out = _run(u_idx, m_idx, user_factors, user_bias.reshape(-1),
               movie_factors, movie_bias.reshape(-1))
    return out.reshape(B, 1)
